# baseline (speedup 1.0000x reference)
import jax
import jax.numpy as jnp
from jax import lax
from jax.experimental import pallas as pl
from jax.experimental.pallas import tpu as pltpu
from jax.experimental.pallas import tpu_sc as plsc

B, N, V = 32, 4, 100000
L = 16
CSZ = 3968
CHUNKS = ([(k * CSZ, CSZ, False) for k in range(25)]
          + [(25 * CSZ, 768, False), (25 * CSZ + 768, 128, True)])
VPAD = 100096
EPS = 1e-10
IMAX = 2147483647


def _lanes():
    return lax.broadcasted_iota(jnp.int32, (L,), 0)


def _bcast_f32(x):
    return jnp.full((L,), x, jnp.float32)


def _mpass(streams, body, carry):
    descs = {}

    def issue(c):
        coff, csz, is_tail = CHUNKS[c]
        par = c % 2
        for s, (src, tsrc, bufs, sems, rows) in enumerate(streams):
            dst = bufs[par].at[pl.ds(0, rows), pl.ds(0, csz)]
            hsrc = tsrc() if is_tail else src(coff, csz)
            descs[(s, c)] = pltpu.async_copy(hsrc, dst, sems[par])

    issue(0)
    for c in range(len(CHUNKS)):
        if c + 1 < len(CHUNKS):
            issue(c + 1)
        for s in range(len(streams)):
            descs[(s, c)].wait()
        coff, csz, _ = CHUNKS[c]
        carry = body([st[2][c % 2] for st in streams], c, coff, csz, carry)
    return carry


def _body(d3, t3, tokp, up, eg3, g23, d_tl, t_tl, eg_tl, g2_tl,
          f_out, i_out, r_scr, q_scr,
          da, db, ta, tb, ga, gb, ra, rb, qa, qb,
          tok_s, u_s, of_s, oi_s,
          sda, sdb, sta, stb, sga, sgb, swa, swb, sqa, sqb):
    wid = lax.axis_index("s") * 2 + lax.axis_index("c")
    b = wid
    lanes = _lanes()
    lane_lt = lanes < N

    pltpu.sync_copy(tokp.at[pl.ds(pl.multiple_of(b * L, 8), L)], tok_s)
    pltpu.sync_copy(up.at[pl.ds(pl.multiple_of(b * L, 8), L)], u_s)
    tok_v = tok_s[...]
    u_v = u_s[...]
    row4 = jnp.where(lane_lt, lanes, 0)

    p1 = [
        (lambda coff, csz: d3.at[b, pl.ds(0, N), pl.ds(coff, csz)],
         lambda: d_tl.at[b, pl.ds(0, N), pl.ds(0, 128)],
         (da, db), (sda, sdb), N),
        (lambda coff, csz: t3.at[b, pl.ds(0, N + 1), pl.ds(coff, csz)],
         lambda: t_tl.at[b, pl.ds(0, N + 1), pl.ds(0, 128)],
         (ta, tb), (sta, stb), N + 1),
        (lambda coff, csz: g23.at[b, pl.ds(0, 1), pl.ds(coff, csz)],
         lambda: g2_tl.at[b, pl.ds(0, 1), pl.ds(0, 128)],
         (ga, gb), (sga, sgb), 1),
    ]

    def body1(cur, c, coff, csz, carry):
        sums, bb, bi, Gd, Gq = carry
        dbuf, tbuf, gbuf = cur

        def it(j, jc):
            sums, bb, bi = jc
            ns = []
            for r in range(N):
                ns.append(sums[r] + jnp.exp(dbuf[r, pl.ds(j * L, L)]))
            for r in range(N):
                ns.append(sums[N + r] + jnp.exp(tbuf[r, pl.ds(j * L, L)]))
            sc = tbuf[N, pl.ds(j * L, L)] + gbuf[0, pl.ds(j * L, L)]
            idx = coff + j * L + lanes
            mk = sc > bb
            bb = jnp.where(mk, sc, bb)
            bi = jnp.where(mk, idx, bi)
            return tuple(ns), bb, bi

        sums, bb, bi = plsc.parallel_loop(0, csz // L, unroll=4,
                                          carry=(sums, bb, bi))(it)

        rel = tok_v - coff
        inr = jnp.logical_and(jnp.logical_and(rel >= 0, rel < csz), lane_lt)
        colg = jnp.where(inr, rel, 0)
        gd = plsc.load_gather(dbuf, [row4, colg])
        gq = plsc.load_gather(tbuf, [row4, colg])
        Gd = jnp.where(inr, gd, Gd)
        Gq = jnp.where(inr, gq, Gq)
        return sums, bb, bi, Gd, Gq

    z = _bcast_f32(0.0)
    sums0 = tuple(z for _ in range(2 * N))
    sums, bb, bi, Gd, Gq = _mpass(
        p1, body1, (sums0, _bcast_f32(-jnp.inf),
                    jnp.zeros((L,), jnp.int32), z, z))

    bv2 = jnp.max(bb)
    bonus = jnp.min(jnp.where(bb == bv2, bi.astype(jnp.float32),
                              jnp.float32(IMAX))).astype(jnp.int32)

    sd_v = z
    sq_v = z
    for r in range(N):
        sel = lanes == r
        sd_v = jnp.where(sel, _bcast_f32(jnp.sum(sums[r])), sd_v)
        sq_v = jnp.where(sel, _bcast_f32(jnp.sum(sums[N + r])), sq_v)

    p_tok = jnp.exp(Gd) / jnp.maximum(sd_v, EPS)
    q_tok = jnp.exp(Gq) / jnp.maximum(sq_v, EPS)
    ap = jnp.minimum(1.0, q_tok / jnp.maximum(p_tok, EPS))
    acc = u_v < ap
    rej = jnp.where(jnp.logical_and(lane_lt, acc),
                    jnp.float32(0.0), jnp.float32(1.0))
    cum = plsc.cumsum(rej)
    am = cum < 0.5
    na = jnp.sum(jnp.where(am, 1.0, 0.0)).astype(jnp.int32)
    fr = jnp.minimum(na, N - 1)
    frv = jnp.full((L,), fr, jnp.int32)

    inv_sd = _bcast_f32(1.0) / _bcast_f32(
        jnp.sum(jnp.where(lanes == fr, sd_v, 0.0)))
    inv_sq = _bcast_f32(1.0) / _bcast_f32(
        jnp.sum(jnp.where(lanes == fr, sq_v, 0.0)))

    p2a = [
        (lambda coff, csz: d3.at[b, pl.ds(0, N), pl.ds(coff, csz)],
         lambda: d_tl.at[b, pl.ds(0, N), pl.ds(0, 128)],
         (da, db), (sda, sdb), N),
        (lambda coff, csz: t3.at[b, pl.ds(0, N), pl.ds(coff, csz)],
         lambda: t_tl.at[b, pl.ds(0, N), pl.ds(0, 128)],
         (ta, tb), (sta, stb), N),
    ]
    wdescs = {}

    def body2a(cur, c, coff, csz, Sv):
        dbuf, tbuf = cur
        par = c % 2
        rbuf = (ra, rb)[par]
        qbuf = (qa, qb)[par]
        if c - 2 >= 0:
            wdescs[("r", c - 2)].wait()
            wdescs[("q", c - 2)].wait()

        def it(j, S):
            col = j * L + lanes
            p = jnp.exp(plsc.load_gather(dbuf, [frv, col])) * inv_sd
            q = jnp.exp(plsc.load_gather(tbuf, [frv, col])) * inv_sq
            res = jnp.maximum(q - p, 0.0)
            rbuf[0, pl.ds(j * L, L)] = res
            qbuf[0, pl.ds(j * L, L)] = q
            return S + res

        Sv = plsc.parallel_loop(0, csz // L, unroll=4, carry=Sv)(it)
        wdescs[("r", c)] = pltpu.async_copy(
            rbuf.at[pl.ds(0, 1), pl.ds(0, csz)],
            r_scr.at[b, pl.ds(0, 1), pl.ds(coff, csz)], swa)
        wdescs[("q", c)] = pltpu.async_copy(
            qbuf.at[pl.ds(0, 1), pl.ds(0, csz)],
            q_scr.at[b, pl.ds(0, 1), pl.ds(coff, csz)], swb)
        return Sv

    Sv = _mpass(p2a, body2a, z)
    nc = len(CHUNKS)
    for c in (nc - 2, nc - 1):
        wdescs[("r", c)].wait()
        wdescs[("q", c)].wait()

    rs = jnp.sum(Sv)
    rs_pos = rs > 0
    inv_rs = _bcast_f32(1.0) / _bcast_f32(jnp.maximum(rs, EPS))

    p2b = [
        (lambda coff, csz: r_scr.at[b, pl.ds(0, 1), pl.ds(coff, csz)],
         lambda: r_scr.at[b, pl.ds(0, 1), pl.ds(V - 32, 128)],
         (da, db), (sda, sdb), 1),
        (lambda coff, csz: q_scr.at[b, pl.ds(0, 1), pl.ds(coff, csz)],
         lambda: q_scr.at[b, pl.ds(0, 1), pl.ds(V - 32, 128)],
         (ta, tb), (sta, stb), 1),
        (lambda coff, csz: eg3.at[b, pl.ds(0, 1), pl.ds(coff, csz)],
         lambda: eg_tl.at[b, pl.ds(0, 1), pl.ds(0, 128)],
         (ga, gb), (sga, sgb), 1),
    ]

    def body2b(cur, c, coff, csz, carry):
        rbuf, qbuf, gbuf = cur

        def it(j, jc):
            bs, bi2 = jc
            res = rbuf[0, pl.ds(j * L, L)]
            q = qbuf[0, pl.ds(j * L, L)]
            cp = jnp.where(rs_pos, res * inv_rs, q)
            score = jnp.maximum(cp, EPS) * gbuf[0, pl.ds(j * L, L)]
            idx = coff + j * L + lanes
            mk = score > bs
            return jnp.where(mk, score, bs), jnp.where(mk, idx, bi2)

        return plsc.parallel_loop(0, csz // L, unroll=4, carry=carry)(it)

    best, besti = _mpass(p2b, body2b,
                         (_bcast_f32(-jnp.inf), jnp.zeros((L,), jnp.int32)))
    bv = jnp.max(best)
    corr = jnp.min(jnp.where(best == bv, besti.astype(jnp.float32),
                             jnp.float32(IMAX))).astype(jnp.int32)

    nxt = jnp.where(na == N, bonus, corr)

    oi = jnp.where(jnp.logical_and(lane_lt, am), tok_v, jnp.int32(0))
    oi = jnp.where(lanes == na, nxt, oi)
    oi = jnp.where(lanes == 5, na, oi)
    oi_s[...] = oi
    of_s[...] = jnp.where(lane_lt, ap, 0.0)
    pltpu.sync_copy(oi_s, i_out.at[pl.ds(pl.multiple_of(b * L, 8), L)])
    pltpu.sync_copy(of_s, f_out.at[pl.ds(pl.multiple_of(b * L, 8), L)])


def _run(d3, t3, tokp, up, eg3, g23, d_tl, t_tl, eg_tl, g2_tl):
    mesh = plsc.VectorSubcoreMesh(core_axis_name="c", subcore_axis_name="s")
    outs = pl.kernel(
        _body,
        out_type=[
            jax.ShapeDtypeStruct((B * L,), jnp.float32),
            jax.ShapeDtypeStruct((B * L,), jnp.int32),
            jax.ShapeDtypeStruct((B, 1, VPAD), jnp.float32),
            jax.ShapeDtypeStruct((B, 1, VPAD), jnp.float32),
        ],
        mesh=mesh,
        compiler_params=pltpu.CompilerParams(needs_layout_passes=False),
        scratch_types=(
            [pltpu.VMEM((N, CSZ), jnp.float32) for _ in range(2)]
            + [pltpu.VMEM((N + 1, CSZ), jnp.float32) for _ in range(2)]
            + [pltpu.VMEM((1, CSZ), jnp.float32) for _ in range(6)]
            + [pltpu.VMEM((L,), jnp.int32),
               pltpu.VMEM((L,), jnp.float32),
               pltpu.VMEM((L,), jnp.float32),
               pltpu.VMEM((L,), jnp.int32)]
            + [pltpu.SemaphoreType.DMA for _ in range(10)]
        ),
    )(d3, t3, tokp, up, eg3, g23, d_tl, t_tl, eg_tl, g2_tl)
    return outs[0], outs[1]


def kernel(draft_logits, target_logits, draft_tokens, u):
    skey = jax.random.key(42)
    eg = jnp.exp(jax.random.gumbel(skey, (B, 1, V), jnp.float32))
    g2 = jax.random.gumbel(jax.random.fold_in(skey, 1), (B, 1, V),
                           jnp.float32)
    NEG = -1e30
    d_tl = jnp.full((B, N, 128), NEG, jnp.float32
                    ).at[:, :, :32].set(draft_logits[:, :, V - 32:])
    t_tl = jnp.full((B, N + 1, 128), NEG, jnp.float32
                    ).at[:, :, :32].set(target_logits[:, :, V - 32:])
    eg_tl = jnp.zeros((B, 1, 128), jnp.float32
                      ).at[:, :, :32].set(eg[:, :, V - 32:])
    g2_tl = jnp.zeros((B, 1, 128), jnp.float32
                      ).at[:, :, :32].set(g2[:, :, V - 32:])
    tokp = jnp.zeros((B, L), jnp.int32).at[:, :N].set(draft_tokens).reshape(-1)
    up = jnp.ones((B, L), jnp.float32).at[:, :N].set(u).reshape(-1)
    f_out, i_out = _run(draft_logits, target_logits, tokp, up, eg, g2,
                        d_tl, t_tl, eg_tl, g2_tl)
    f2 = f_out.reshape(B, L)
    i2 = i_out.reshape(B, L)
    out_tokens = i2[:, :N + 1]
    accept_prob = f2[:, :N]
    num_accepted = i2[:, 5]
    return out_tokens, accept_prob, num_accepted


# speedup vs baseline: 1.0171x; 1.0171x over previous
import jax
import jax.numpy as jnp
from jax import lax
from jax.experimental import pallas as pl
from jax.experimental.pallas import tpu as pltpu
from jax.experimental.pallas import tpu_sc as plsc

B, N, V = 32, 4, 100000
L = 16
CSZ = 3968
CHUNKS = ([(k * CSZ, CSZ, False) for k in range(25)]
          + [(25 * CSZ, 768, False), (25 * CSZ + 768, 128, True)])
EPS = 1e-10
IMAX = 2147483647


def _lanes():
    return lax.broadcasted_iota(jnp.int32, (L,), 0)


def _bcast_f32(x):
    return jnp.full((L,), x, jnp.float32)


def _mpass(streams, body, carry):
    descs = {}

    def issue(c):
        coff, csz, is_tail = CHUNKS[c]
        par = c % 2
        for s, (src, tsrc, bufs, sems, rows) in enumerate(streams):
            dst = bufs[par].at[pl.ds(0, rows), pl.ds(0, csz)]
            hsrc = tsrc() if is_tail else src(coff, csz)
            descs[(s, c)] = pltpu.async_copy(hsrc, dst, sems[par])

    issue(0)
    for c in range(len(CHUNKS)):
        if c + 1 < len(CHUNKS):
            issue(c + 1)
        for s in range(len(streams)):
            descs[(s, c)].wait()
        coff, csz, _ = CHUNKS[c]
        carry = body([st[2][c % 2] for st in streams], c, coff, csz, carry)
    return carry


def _body(d3, t3, tokp, up, eg3, g23, d_tl, t_tl, eg_tl, g2_tl,
          f_out, i_out,
          da, db, ta, tb, ga, gb,
          tok_s, u_s, of_s, oi_s,
          sda, sdb, sta, stb, sga, sgb):
    wid = lax.axis_index("s") * 2 + lax.axis_index("c")
    b = wid
    lanes = _lanes()
    lane_lt = lanes < N

    pltpu.sync_copy(tokp.at[pl.ds(pl.multiple_of(b * L, 8), L)], tok_s)
    pltpu.sync_copy(up.at[pl.ds(pl.multiple_of(b * L, 8), L)], u_s)
    tok_v = tok_s[...]
    u_v = u_s[...]
    row4 = jnp.where(lane_lt, lanes, 0)

    p1 = [
        (lambda coff, csz: d3.at[b, pl.ds(0, N), pl.ds(coff, csz)],
         lambda: d_tl.at[b, pl.ds(0, N), pl.ds(0, 128)],
         (da, db), (sda, sdb), N),
        (lambda coff, csz: t3.at[b, pl.ds(0, N + 1), pl.ds(coff, csz)],
         lambda: t_tl.at[b, pl.ds(0, N + 1), pl.ds(0, 128)],
         (ta, tb), (sta, stb), N + 1),
        (lambda coff, csz: g23.at[b, pl.ds(0, 1), pl.ds(coff, csz)],
         lambda: g2_tl.at[b, pl.ds(0, 1), pl.ds(0, 128)],
         (ga, gb), (sga, sgb), 1),
    ]

    def body1(cur, c, coff, csz, carry):
        sums, bb, bi, Gd, Gq = carry
        dbuf, tbuf, gbuf = cur

        def it(j, jc):
            sums, bb, bi = jc
            ns = []
            for r in range(N):
                ns.append(sums[r] + jnp.exp(dbuf[r, pl.ds(j * L, L)]))
            for r in range(N):
                ns.append(sums[N + r] + jnp.exp(tbuf[r, pl.ds(j * L, L)]))
            sc = tbuf[N, pl.ds(j * L, L)] + gbuf[0, pl.ds(j * L, L)]
            idx = coff + j * L + lanes
            mk = sc > bb
            bb = jnp.where(mk, sc, bb)
            bi = jnp.where(mk, idx, bi)
            return tuple(ns), bb, bi

        sums, bb, bi = plsc.parallel_loop(0, csz // L, unroll=4,
                                          carry=(sums, bb, bi))(it)

        rel = tok_v - coff
        inr = jnp.logical_and(jnp.logical_and(rel >= 0, rel < csz), lane_lt)
        colg = jnp.where(inr, rel, 0)
        gd = plsc.load_gather(dbuf, [row4, colg])
        gq = plsc.load_gather(tbuf, [row4, colg])
        Gd = jnp.where(inr, gd, Gd)
        Gq = jnp.where(inr, gq, Gq)
        return sums, bb, bi, Gd, Gq

    z = _bcast_f32(0.0)
    sums0 = tuple(z for _ in range(2 * N))
    sums, bb, bi, Gd, Gq = _mpass(
        p1, body1, (sums0, _bcast_f32(-jnp.inf),
                    jnp.zeros((L,), jnp.int32), z, z))

    bv2 = jnp.max(bb)
    bonus = jnp.min(jnp.where(bb == bv2, bi.astype(jnp.float32),
                              jnp.float32(IMAX))).astype(jnp.int32)

    sd_v = z
    sq_v = z
    for r in range(N):
        sel = lanes == r
        sd_v = jnp.where(sel, _bcast_f32(jnp.sum(sums[r])), sd_v)
        sq_v = jnp.where(sel, _bcast_f32(jnp.sum(sums[N + r])), sq_v)

    p_tok = jnp.exp(Gd) / jnp.maximum(sd_v, EPS)
    q_tok = jnp.exp(Gq) / jnp.maximum(sq_v, EPS)
    ap = jnp.minimum(1.0, q_tok / jnp.maximum(p_tok, EPS))
    acc = u_v < ap
    rej = jnp.where(jnp.logical_and(lane_lt, acc),
                    jnp.float32(0.0), jnp.float32(1.0))
    cum = plsc.cumsum(rej)
    am = cum < 0.5
    na = jnp.sum(jnp.where(am, 1.0, 0.0)).astype(jnp.int32)
    fr = jnp.minimum(na, N - 1)
    frv = jnp.full((L,), fr, jnp.int32)

    inv_sd = _bcast_f32(1.0) / _bcast_f32(
        jnp.sum(jnp.where(lanes == fr, sd_v, 0.0)))
    inv_sq = _bcast_f32(1.0) / _bcast_f32(
        jnp.sum(jnp.where(lanes == fr, sq_v, 0.0)))

    p2 = [
        (lambda coff, csz: d3.at[b, pl.ds(0, N), pl.ds(coff, csz)],
         lambda: d_tl.at[b, pl.ds(0, N), pl.ds(0, 128)],
         (da, db), (sda, sdb), N),
        (lambda coff, csz: t3.at[b, pl.ds(0, N), pl.ds(coff, csz)],
         lambda: t_tl.at[b, pl.ds(0, N), pl.ds(0, 128)],
         (ta, tb), (sta, stb), N),
        (lambda coff, csz: eg3.at[b, pl.ds(0, 1), pl.ds(coff, csz)],
         lambda: eg_tl.at[b, pl.ds(0, 1), pl.ds(0, 128)],
         (ga, gb), (sga, sgb), 1),
    ]

    def body2(cur, c, coff, csz, carry):
        dbuf, tbuf, gbuf = cur

        def it(j, jc):
            S, Av, Ai, Qv, Qi, Ev, Ei = jc
            col = j * L + lanes
            p = jnp.exp(plsc.load_gather(dbuf, [frv, col])) * inv_sd
            q = jnp.exp(plsc.load_gather(tbuf, [frv, col])) * inv_sq
            res = jnp.maximum(q - p, 0.0)
            e = gbuf[0, pl.ds(j * L, L)]
            idx = coff + j * L + lanes
            a = res * e
            qe = q * e
            m1 = a > Av
            Av = jnp.where(m1, a, Av)
            Ai = jnp.where(m1, idx, Ai)
            m2 = qe > Qv
            Qv = jnp.where(m2, qe, Qv)
            Qi = jnp.where(m2, idx, Qi)
            m3 = e > Ev
            Ev = jnp.where(m3, e, Ev)
            Ei = jnp.where(m3, idx, Ei)
            return S + res, Av, Ai, Qv, Qi, Ev, Ei

        return plsc.parallel_loop(0, csz // L, unroll=4, carry=carry)(it)

    zi = jnp.zeros((L,), jnp.int32)
    ninf = _bcast_f32(-jnp.inf)
    S, Av, Ai, Qv, Qi, Ev, Ei = _mpass(
        p2, body2, (z, ninf, zi, ninf, zi, ninf, zi))

    def _pick(vv, ii):
        m = jnp.max(vv)
        return m, jnp.min(jnp.where(vv == m, ii.astype(jnp.float32),
                                    jnp.float32(IMAX)))

    Am, Aidx = _pick(Av, Ai)
    Qm, Qidx = _pick(Qv, Qi)
    Em, Eidx = _pick(Ev, Ei)

    rs = jnp.sum(S)
    rs_pos = rs > 0
    inv_rs = _bcast_f32(1.0) / _bcast_f32(jnp.maximum(rs, EPS))
    c1v = jnp.where(rs_pos, _bcast_f32(Am) * inv_rs, _bcast_f32(Qm))
    c2v = _bcast_f32(Em) * _bcast_f32(EPS)
    c1i = jnp.where(rs_pos, _bcast_f32(Aidx), _bcast_f32(Qidx))
    ei_v = _bcast_f32(Eidx)
    ci_v = jnp.where(c1v > c2v, c1i,
                     jnp.where(c1v < c2v, ei_v, jnp.minimum(c1i, ei_v)))
    corr = jnp.max(ci_v).astype(jnp.int32)

    nxt = jnp.where(na == N, bonus, corr)

    oi = jnp.where(jnp.logical_and(lane_lt, am), tok_v, jnp.int32(0))
    oi = jnp.where(lanes == na, nxt, oi)
    oi = jnp.where(lanes == 5, na, oi)
    oi_s[...] = oi
    of_s[...] = jnp.where(lane_lt, ap, 0.0)
    pltpu.sync_copy(oi_s, i_out.at[pl.ds(pl.multiple_of(b * L, 8), L)])
    pltpu.sync_copy(of_s, f_out.at[pl.ds(pl.multiple_of(b * L, 8), L)])


def _run(d3, t3, tokp, up, eg3, g23, d_tl, t_tl, eg_tl, g2_tl):
    mesh = plsc.VectorSubcoreMesh(core_axis_name="c", subcore_axis_name="s")
    outs = pl.kernel(
        _body,
        out_type=[
            jax.ShapeDtypeStruct((B * L,), jnp.float32),
            jax.ShapeDtypeStruct((B * L,), jnp.int32),
        ],
        mesh=mesh,
        compiler_params=pltpu.CompilerParams(needs_layout_passes=False),
        scratch_types=(
            [pltpu.VMEM((N, CSZ), jnp.float32) for _ in range(2)]
            + [pltpu.VMEM((N + 1, CSZ), jnp.float32) for _ in range(2)]
            + [pltpu.VMEM((1, CSZ), jnp.float32) for _ in range(2)]
            + [pltpu.VMEM((L,), jnp.int32),
               pltpu.VMEM((L,), jnp.float32),
               pltpu.VMEM((L,), jnp.float32),
               pltpu.VMEM((L,), jnp.int32)]
            + [pltpu.SemaphoreType.DMA for _ in range(6)]
        ),
    )(d3, t3, tokp, up, eg3, g23, d_tl, t_tl, eg_tl, g2_tl)
    return outs[0], outs[1]


def kernel(draft_logits, target_logits, draft_tokens, u):
    skey = jax.random.key(42)
    eg = jnp.exp(jax.random.gumbel(skey, (B, 1, V), jnp.float32))
    g2 = jax.random.gumbel(jax.random.fold_in(skey, 1), (B, 1, V),
                           jnp.float32)
    NEG = -1e30
    d_tl = jnp.full((B, N, 128), NEG, jnp.float32
                    ).at[:, :, :32].set(draft_logits[:, :, V - 32:])
    t_tl = jnp.full((B, N + 1, 128), NEG, jnp.float32
                    ).at[:, :, :32].set(target_logits[:, :, V - 32:])
    eg_tl = jnp.zeros((B, 1, 128), jnp.float32
                      ).at[:, :, :32].set(eg[:, :, V - 32:])
    g2_tl = jnp.zeros((B, 1, 128), jnp.float32
                      ).at[:, :, :32].set(g2[:, :, V - 32:])
    tokp = jnp.zeros((B, L), jnp.int32).at[:, :N].set(draft_tokens).reshape(-1)
    up = jnp.ones((B, L), jnp.float32).at[:, :N].set(u).reshape(-1)
    f_out, i_out = _run(draft_logits, target_logits, tokp, up, eg, g2,
                        d_tl, t_tl, eg_tl, g2_tl)
    f2 = f_out.reshape(B, L)
    i2 = i_out.reshape(B, L)
    out_tokens = i2[:, :N + 1]
    accept_prob = f2[:, :N]
    num_accepted = i2[:, 5]
    return out_tokens, accept_prob, num_accepted
